# Initial kernel scaffold; baseline (speedup 1.0000x reference)
import jax
import jax.numpy as jnp
from jax.experimental import pallas as pl


def kernel(inputs, token_type_ids, word_table, type_table, pos_table, ln_gamma, ln_beta):
    raise NotImplementedError("write your pallas kernel here")


# speedup vs baseline: 1.9421x; 1.9421x over previous
import functools

import jax
import jax.numpy as jnp
from jax import lax
from jax.experimental import pallas as pl
from jax.experimental.pallas import tpu as pltpu
from jax.experimental.pallas import tpu_sc as plsc

EPS = 1e-3

_NC = 2
_NS = 16
_NW = _NC * _NS

_G = 64


def _sc_gather(word_table, ids_flat, n_tokens, d):
  per_w = n_tokens // _NW
  ch = per_w // _G
  ids_resh = ids_flat.reshape(_NW, ch, _G)

  mesh = plsc.VectorSubcoreMesh(core_axis_name="c", subcore_axis_name="s")

  @functools.partial(
      pl.kernel,
      mesh=mesh,
      out_type=jax.ShapeDtypeStruct((n_tokens, d), jnp.float32),
      scratch_types=[
          pltpu.VMEM((ch, _G), jnp.int32),
          pltpu.VMEM((_G, d), jnp.float32),
          pltpu.VMEM((_G, d), jnp.float32),
          pltpu.SemaphoreType.DMA,
          pltpu.SemaphoreType.DMA,
          pltpu.SemaphoreType.DMA,
          pltpu.SemaphoreType.DMA,
      ],
  )
  def gather_kernel(table_hbm, ids_hbm, out_hbm, idx_v, rows0, rows1,
                    g0, g1, o0, o1):
    wid = lax.axis_index("s") * _NC + lax.axis_index("c")
    base = wid * per_w
    pltpu.sync_copy(ids_hbm.at[wid], idx_v)
    rows = (rows0, rows1)
    gsem = (g0, g1)
    osem = (o0, o1)
    gh = [None, None]
    oh = [None, None]
    gh[0] = pltpu.async_copy(table_hbm.at[idx_v.at[0]], rows[0], gsem[0])
    for c in range(ch):
      b = c & 1
      nb = 1 - b
      if c + 1 < ch:
        if oh[nb] is not None:
          oh[nb].wait()
        gh[nb] = pltpu.async_copy(
            table_hbm.at[idx_v.at[c + 1]], rows[nb], gsem[nb])
      gh[b].wait()
      oh[b] = pltpu.async_copy(
          rows[b], out_hbm.at[pl.ds(base + c * _G, _G)], osem[b])
    for h in oh:
      if h is not None:
        h.wait()

  return gather_kernel(word_table, ids_resh)


def _ln_body(g_ref, tt_ref, ty_ref, pos_ref, gam_ref, bet_ref, out_ref):
  x = g_ref[0]
  ttc = tt_ref[0]
  t0 = ty_ref[0:1, :]
  t1 = ty_ref[1:2, :]
  e = x + t0 + ttc * (t1 - t0) + pos_ref[...]
  m = jnp.mean(e, axis=-1, keepdims=True)
  dlt = e - m
  v = jnp.mean(dlt * dlt, axis=-1, keepdims=True)
  y = dlt * lax.rsqrt(v + EPS)
  out_ref[0] = y * gam_ref[...] + bet_ref[...]


def _tc_ln(gathered, ttf, type2, pos, gamma2d, beta2d):
  b, s, d = gathered.shape
  return pl.pallas_call(
      _ln_body,
      grid=(b,),
      in_specs=[
          pl.BlockSpec((1, s, d), lambda i: (i, 0, 0)),
          pl.BlockSpec((1, s, 1), lambda i: (i, 0, 0)),
          pl.BlockSpec((2, d), lambda i: (0, 0)),
          pl.BlockSpec((s, d), lambda i: (0, 0)),
          pl.BlockSpec((1, d), lambda i: (0, 0)),
          pl.BlockSpec((1, d), lambda i: (0, 0)),
      ],
      out_specs=pl.BlockSpec((1, s, d), lambda i: (i, 0, 0)),
      out_shape=jax.ShapeDtypeStruct((b, s, d), jnp.float32),
  )(gathered, ttf, type2, pos, gamma2d, beta2d)


def kernel(inputs, token_type_ids, word_table, type_table, pos_table,
           ln_gamma, ln_beta):
  b, s = inputs.shape
  v, d = word_table.shape
  n_tokens = b * s

  gathered = _sc_gather(word_table, inputs.reshape(-1), n_tokens, d)

  ttf = token_type_ids.astype(jnp.float32).reshape(b, s, 1)
  out = _tc_ln(
      gathered.reshape(b, s, d),
      ttf,
      type_table[:2],
      pos_table[:s],
      ln_gamma.reshape(1, d),
      ln_beta.reshape(1, d),
  )
  return out


# speedup vs baseline: 2.0636x; 1.0625x over previous
import functools

import jax
import jax.numpy as jnp
from jax import lax
from jax.experimental import pallas as pl
from jax.experimental.pallas import tpu as pltpu
from jax.experimental.pallas import tpu_sc as plsc

EPS = 1e-3

_NC = 2
_NS = 16
_NW = _NC * _NS

_G = 64


def _sc_gather(word_table, ids_flat, n_tokens, d):
  per_w = n_tokens // _NW
  ch = per_w // _G
  ids_resh = ids_flat.reshape(_NW, ch, _G)

  mesh = plsc.VectorSubcoreMesh(core_axis_name="c", subcore_axis_name="s")

  @functools.partial(
      pl.kernel,
      mesh=mesh,
      out_type=jax.ShapeDtypeStruct((n_tokens, d), jnp.float32),
      scratch_types=[
          pltpu.VMEM((ch, _G), jnp.int32),
          pltpu.VMEM((_G, d), jnp.float32),
          pltpu.VMEM((_G, d), jnp.float32),
          pltpu.SemaphoreType.DMA,
          pltpu.SemaphoreType.DMA,
          pltpu.SemaphoreType.DMA,
          pltpu.SemaphoreType.DMA,
      ],
  )
  def gather_kernel(table_hbm, ids_hbm, out_hbm, idx_v, rows0, rows1,
                    g0, g1, o0, o1):
    wid = lax.axis_index("s") * _NC + lax.axis_index("c")
    base = wid * per_w
    pltpu.sync_copy(ids_hbm.at[wid], idx_v)
    rows = (rows0, rows1)
    gsem = (g0, g1)
    osem = (o0, o1)
    gh = [None, None]
    oh = [None, None]
    gh[0] = pltpu.async_copy(table_hbm.at[idx_v.at[0]], rows[0], gsem[0])
    for c in range(ch):
      b = c & 1
      nb = 1 - b
      if c + 1 < ch:
        if oh[nb] is not None:
          oh[nb].wait()
        gh[nb] = pltpu.async_copy(
            table_hbm.at[idx_v.at[c + 1]], rows[nb], gsem[nb])
      gh[b].wait()
      oh[b] = pltpu.async_copy(
          rows[b], out_hbm.at[pl.ds(base + c * _G, _G)], osem[b])
    for h in oh:
      if h is not None:
        h.wait()

  return gather_kernel(word_table, ids_resh)


def _ln_body(g_ref, tt_ref, ty_ref, pos_ref, gam_ref, bet_ref, out_ref):
  x = g_ref[0]
  ttc = tt_ref[0]
  t0 = ty_ref[0:1, :]
  t1 = ty_ref[1:2, :]
  e = x + t0 + ttc * (t1 - t0) + pos_ref[...]
  m = jnp.mean(e, axis=-1, keepdims=True)
  dlt = e - m
  v = jnp.mean(dlt * dlt, axis=-1, keepdims=True)
  y = dlt * lax.rsqrt(v + EPS)
  out_ref[0] = y * gam_ref[...] + bet_ref[...]


def _ln_body_aliased(prev_ref, g_ref, tt_ref, ty_ref, pos_ref, gam_ref,
                     bet_ref, out_ref):
  del prev_ref
  _ln_body(g_ref, tt_ref, ty_ref, pos_ref, gam_ref, bet_ref, out_ref)


def _tc_ln_chunk(prev, gathered, ttf, type2, pos, gamma2d, beta2d,
                 b_total, base_blk):
  nb, s, d = gathered.shape
  common_in = [
      pl.BlockSpec((1, s, d), lambda i: (i, 0, 0)),
      pl.BlockSpec((1, s, 1), lambda i: (i, 0, 0)),
      pl.BlockSpec((2, d), lambda i: (0, 0)),
      pl.BlockSpec((s, d), lambda i: (0, 0)),
      pl.BlockSpec((1, d), lambda i: (0, 0)),
      pl.BlockSpec((1, d), lambda i: (0, 0)),
  ]
  out_spec = pl.BlockSpec((1, s, d), lambda i: (base_blk + i, 0, 0))
  out_shape = jax.ShapeDtypeStruct((b_total, s, d), jnp.float32)
  if prev is None:
    return pl.pallas_call(
        _ln_body,
        grid=(nb,),
        in_specs=common_in,
        out_specs=out_spec,
        out_shape=out_shape,
    )(gathered, ttf, type2, pos, gamma2d, beta2d)
  return pl.pallas_call(
      _ln_body_aliased,
      grid=(nb,),
      in_specs=[pl.BlockSpec(memory_space=pl.ANY)] + common_in,
      out_specs=out_spec,
      out_shape=out_shape,
      input_output_aliases={0: 0},
  )(prev, gathered, ttf, type2, pos, gamma2d, beta2d)


_K = 4


def kernel(inputs, token_type_ids, word_table, type_table, pos_table,
           ln_gamma, ln_beta):
  b, s = inputs.shape
  v, d = word_table.shape

  bc = b // _K
  n_tok_c = bc * s
  ids = inputs.reshape(_K, n_tok_c)
  ttf = token_type_ids.astype(jnp.float32).reshape(_K, bc, s, 1)
  type2 = type_table[:2]
  pos = pos_table[:s]
  gamma2d = ln_gamma.reshape(1, d)
  beta2d = ln_beta.reshape(1, d)

  out = None
  for k in range(_K):
    g_k = _sc_gather(word_table, ids[k], n_tok_c, d)
    out = _tc_ln_chunk(out, g_k.reshape(bc, s, d), ttf[k], type2, pos,
                       gamma2d, beta2d, b, k * bc)
  return out
